# baseline (device time: 15800 ns/iter reference)
import jax
from jax import lax
from jax.experimental import pallas as pl
from jax.experimental.pallas import tpu as pltpu

N_Y = 4
N_X = 2
N_P = 2


def kernel(x):
    x2 = x.reshape(x.shape[1], x.shape[2])
    m, n_total = x2.shape
    n_chunk = n_total // N_Y
    n_piece = n_chunk // N_P
    m_half = m // N_X

    def body(x_ref, out_ref, recv_buf, send_sems, recv_sems,
             send_sems_x, recv_sems_x):
        my_x = lax.axis_index("x")
        my_y = lax.axis_index("y")
        my_z = lax.axis_index("z")
        partner_x = 1 - my_x
        row0 = my_x * m_half
        prow0 = partner_x * m_half

        barrier_sem = pltpu.get_barrier_semaphore()
        for k in range(1, N_Y):
            peer = lax.rem(my_y + k, N_Y)
            pl.semaphore_signal(
                barrier_sem, inc=1,
                device_id=(my_x, peer, my_z),
                device_id_type=pl.DeviceIdType.MESH,
            )
        pl.semaphore_signal(
            barrier_sem, inc=1,
            device_id=(partner_x, my_y, my_z),
            device_id_type=pl.DeviceIdType.MESH,
        )
        pl.semaphore_wait(barrier_sem, N_Y)

        sends = []
        for p in range(N_P):
            for k in range(1, N_Y):
                dst = lax.rem(my_y + k, N_Y)
                slot = N_Y - 1 - k
                rdma = pltpu.make_async_remote_copy(
                    src_ref=x_ref.at[pl.ds(row0, m_half),
                                     pl.ds(dst * n_chunk + p * n_piece,
                                           n_piece)],
                    dst_ref=recv_buf.at[slot, :, pl.ds(p * n_piece, n_piece)],
                    send_sem=send_sems.at[k - 1, p],
                    recv_sem=recv_sems.at[slot, p],
                    device_id=(my_x, dst, my_z),
                    device_id_type=pl.DeviceIdType.MESH,
                )
                rdma.start()
                sends.append(rdma)

        xchgs = []
        for p in range(N_P):
            for slot in range(N_Y - 1):
                recv = pltpu.make_async_remote_copy(
                    src_ref=recv_buf.at[slot, :, pl.ds(p * n_piece, n_piece)],
                    dst_ref=recv_buf.at[slot, :, pl.ds(p * n_piece, n_piece)],
                    send_sem=send_sems.at[slot, p],
                    recv_sem=recv_sems.at[slot, p],
                    device_id=(my_x, my_y, my_z),
                    device_id_type=pl.DeviceIdType.MESH,
                )
                recv.wait_recv()

            cols = pl.ds(p * n_piece, n_piece)
            out_ref[pl.ds(row0, m_half), cols] = (
                x_ref[pl.ds(row0, m_half),
                      pl.ds(my_y * n_chunk + p * n_piece, n_piece)]
                + recv_buf[0, :, p * n_piece:(p + 1) * n_piece]
                + recv_buf[1, :, p * n_piece:(p + 1) * n_piece]
                + recv_buf[2, :, p * n_piece:(p + 1) * n_piece]
            )

            xchg = pltpu.make_async_remote_copy(
                src_ref=out_ref.at[pl.ds(row0, m_half), cols],
                dst_ref=out_ref.at[pl.ds(row0, m_half), cols],
                send_sem=send_sems_x.at[p],
                recv_sem=recv_sems_x.at[p],
                device_id=(partner_x, my_y, my_z),
                device_id_type=pl.DeviceIdType.MESH,
            )
            xchg.start()
            xchgs.append(xchg)

        for p in range(N_P):
            cols = pl.ds(p * n_piece, n_piece)
            xrecv = pltpu.make_async_remote_copy(
                src_ref=out_ref.at[pl.ds(prow0, m_half), cols],
                dst_ref=out_ref.at[pl.ds(prow0, m_half), cols],
                send_sem=send_sems_x.at[p],
                recv_sem=recv_sems_x.at[p],
                device_id=(partner_x, my_y, my_z),
                device_id_type=pl.DeviceIdType.MESH,
            )
            xrecv.wait_recv()
        for xchg in xchgs:
            xchg.wait_send()
        for rdma in sends:
            rdma.wait_send()

    return pl.pallas_call(
        body,
        out_shape=jax.ShapeDtypeStruct((m, n_chunk), x2.dtype),
        in_specs=[pl.BlockSpec(memory_space=pltpu.VMEM)],
        out_specs=pl.BlockSpec(memory_space=pltpu.VMEM),
        scratch_shapes=[
            pltpu.VMEM((N_Y - 1, m_half, n_chunk), x2.dtype),
            pltpu.SemaphoreType.DMA((N_Y - 1, N_P)),
            pltpu.SemaphoreType.DMA((N_Y - 1, N_P)),
            pltpu.SemaphoreType.DMA((N_P,)),
            pltpu.SemaphoreType.DMA((N_P,)),
        ],
        compiler_params=pltpu.CompilerParams(collective_id=0),
    )(x2)
